# Initial kernel scaffold; baseline (speedup 1.0000x reference)
import jax
import jax.numpy as jnp
from jax.experimental import pallas as pl


def kernel(x, edge_index, W1, b1, W2, b2, Ws, bs, gs, betas, Wf, bf, gf, bf2):
    raise NotImplementedError("write your pallas kernel here")


# speedup vs baseline: 2.9148x; 2.9148x over previous
import functools

import jax
import jax.numpy as jnp
from jax.experimental import pallas as pl
from jax.experimental.pallas import tpu as pltpu

_EPS = 1e-5


def _fused_kernel(x_ref, w1_ref, b1_ref, w2p_ref, b2p_ref,
                  wcat_ref, bcat_ref, gcat_ref, betacat_ref,
                  wf_ref, bf_ref, gf_ref, bf2_ref,
                  out_ref, acc_ref, sw_ref, *, n_rows, n_blocks, n_strategies, d):
    t = pl.program_id(0)
    i = pl.program_id(1)

    @pl.when(t == 0)
    def _pass0():
        part = jnp.sum(x_ref[...], axis=0, keepdims=True)

        @pl.when(i == 0)
        def _():
            acc_ref[...] = part

        @pl.when(i > 0)
        def _():
            acc_ref[...] = acc_ref[...] + part

        @pl.when(i == n_blocks - 1)
        def _():
            gr = acc_ref[...] * (1.0 / n_rows)
            h = jnp.maximum(
                jnp.dot(gr, w1_ref[...], preferred_element_type=jnp.float32)
                + b1_ref[...], 0.0)
            logits = (jnp.dot(h, w2p_ref[...], preferred_element_type=jnp.float32)
                      + b2p_ref[...])
            m = jnp.max(logits, axis=-1, keepdims=True)
            e = jnp.exp(logits - m)
            sw_ref[...] = e / jnp.sum(e, axis=-1, keepdims=True)

    @pl.when(t == 1)
    def _pass1():
        xb = x_ref[...]
        h = (jnp.dot(xb, wcat_ref[...], preferred_element_type=jnp.float32)
             + bcat_ref[...])
        acc = jnp.zeros_like(xb)
        inv_d = 1.0 / d
        for s in range(n_strategies):
            sl = slice(s * d, (s + 1) * d)
            hs = h[:, sl]
            ss = jnp.sum(hs * hs, axis=-1, keepdims=True) * inv_d
            r = jax.lax.rsqrt(ss + _EPS)
            sw_s = sw_ref[0:1, s:s + 1]
            g = gcat_ref[:, sl] * sw_s
            beta = betacat_ref[:, sl] * sw_s
            acc = acc + jnp.maximum(hs * r * g + beta, 0.0)
        y = (jnp.dot(acc, wf_ref[...], preferred_element_type=jnp.float32)
             + bf_ref[...])
        ss = jnp.sum(y * y, axis=-1, keepdims=True) * inv_d
        r = jax.lax.rsqrt(ss + _EPS)
        out_ref[...] = jnp.maximum(y * r * gf_ref[...] + bf2_ref[...], 0.0)


def kernel(x, edge_index, W1, b1, W2, b2, Ws, bs, gs, betas, Wf, bf, gf, bf2):
    del edge_index
    n, d = x.shape
    s = Ws.shape[0]
    dh = W1.shape[1]
    lanes = 128

    w2p = jnp.zeros((dh, lanes), W2.dtype).at[:, :s].set(W2)
    b2p = jnp.full((1, lanes), -1e30, b2.dtype).at[0, :s].set(b2)
    wsc = Ws - jnp.mean(Ws, axis=2, keepdims=True)
    bsc = bs - jnp.mean(bs, axis=1, keepdims=True)
    wfc = Wf - jnp.mean(Wf, axis=1, keepdims=True)
    bfc = bf - jnp.mean(bf)
    wcat = jnp.transpose(wsc, (1, 0, 2)).reshape(d, s * d)
    bcat = bsc.reshape(1, s * d)
    gcat = gs.reshape(1, s * d)
    betacat = betas.reshape(1, s * d)

    blk = 1000
    assert n % blk == 0
    nb = n // blk
    const = lambda t, i: (0, 0)
    out = pl.pallas_call(
        functools.partial(_fused_kernel, n_rows=float(n), n_blocks=nb,
                          n_strategies=s, d=d),
        grid=(2, nb),
        in_specs=[
            pl.BlockSpec((blk, d), lambda t, i: (i, 0)),
            pl.BlockSpec((d, dh), const),
            pl.BlockSpec((1, dh), const),
            pl.BlockSpec((dh, lanes), const),
            pl.BlockSpec((1, lanes), const),
            pl.BlockSpec((d, s * d), const),
            pl.BlockSpec((1, s * d), const),
            pl.BlockSpec((1, s * d), const),
            pl.BlockSpec((1, s * d), const),
            pl.BlockSpec((d, d), const),
            pl.BlockSpec((1, d), const),
            pl.BlockSpec((1, d), const),
            pl.BlockSpec((1, d), const),
        ],
        out_specs=pl.BlockSpec((blk, d),
                               lambda t, i: (jnp.where(t == 0, 0, i), 0)),
        out_shape=jax.ShapeDtypeStruct((n, d), x.dtype),
        scratch_shapes=[
            pltpu.VMEM((1, lanes), jnp.float32),
            pltpu.VMEM((1, lanes), jnp.float32),
        ],
        compiler_params=pltpu.CompilerParams(
            dimension_semantics=("arbitrary", "arbitrary"),
        ),
    )(x, W1, b1.reshape(1, dh), w2p, b2p, wcat, bcat, gcat, betacat,
      wfc, bfc.reshape(1, d), gf.reshape(1, d), bf2.reshape(1, d))
    return out


# speedup vs baseline: 2.9443x; 1.0101x over previous
import functools

import jax
import jax.numpy as jnp
from jax.experimental import pallas as pl
from jax.experimental.pallas import tpu as pltpu

_EPS = 1e-5


def _selector_kernel(x_ref, w1_ref, b1_ref, w2p_ref, b2p_ref, sw_ref,
                     *, n_rows, n_blocks):
    i = pl.program_id(0)
    part = jnp.sum(x_ref[...], axis=0, keepdims=True)

    @pl.when(i == 0)
    def _():
        sw_ref[...] = part

    @pl.when(i > 0)
    def _():
        sw_ref[...] = sw_ref[...] + part

    @pl.when(i == n_blocks - 1)
    def _():
        gr = sw_ref[...] * (1.0 / n_rows)
        h = jnp.maximum(
            jnp.dot(gr, w1_ref[...], preferred_element_type=jnp.float32)
            + b1_ref[...], 0.0)
        logits = (jnp.dot(h, w2p_ref[...], preferred_element_type=jnp.float32)
                  + b2p_ref[...])
        m = jnp.max(logits, axis=-1, keepdims=True)
        e = jnp.exp(logits - m)
        sw_ref[...] = e / jnp.sum(e, axis=-1, keepdims=True)


def _main_kernel(x_ref, sw_ref, wcat_ref, bcat_ref, gcat_ref, betacat_ref,
                 wf_ref, bf_ref, gf_ref, bf2_ref, out_ref,
                 *, n_strategies, d):
    xb = x_ref[...]
    h = (jnp.dot(xb, wcat_ref[...], preferred_element_type=jnp.float32)
         + bcat_ref[...])
    acc = jnp.zeros_like(xb)
    inv_d = 1.0 / d
    for s in range(n_strategies):
        sl = slice(s * d, (s + 1) * d)
        hs = h[:, sl]
        ss = jnp.sum(hs * hs, axis=-1, keepdims=True) * inv_d
        r = jax.lax.rsqrt(ss + _EPS)
        sw_s = sw_ref[0:1, s:s + 1]
        g = gcat_ref[:, sl] * sw_s
        beta = betacat_ref[:, sl] * sw_s
        acc = acc + jnp.maximum(hs * r * g + beta, 0.0)
    y = (jnp.dot(acc, wf_ref[...], preferred_element_type=jnp.float32)
         + bf_ref[...])
    ss = jnp.sum(y * y, axis=-1, keepdims=True) * inv_d
    r = jax.lax.rsqrt(ss + _EPS)
    out_ref[...] = jnp.maximum(y * r * gf_ref[...] + bf2_ref[...], 0.0)


def kernel(x, edge_index, W1, b1, W2, b2, Ws, bs, gs, betas, Wf, bf, gf, bf2):
    del edge_index
    n, d = x.shape
    s = Ws.shape[0]
    dh = W1.shape[1]
    lanes = 128

    w2p = jnp.zeros((dh, lanes), W2.dtype).at[:, :s].set(W2)
    b2p = jnp.full((1, lanes), -1e30, b2.dtype).at[0, :s].set(b2)
    wsc = Ws - jnp.mean(Ws, axis=2, keepdims=True)
    bsc = bs - jnp.mean(bs, axis=1, keepdims=True)
    wfc = Wf - jnp.mean(Wf, axis=1, keepdims=True)
    bfc = (bf - jnp.mean(bf)).reshape(1, d)
    wcat = jnp.transpose(wsc, (1, 0, 2)).reshape(d, s * d)
    bcat = bsc.reshape(1, s * d)
    gcat = gs.reshape(1, s * d)
    betacat = betas.reshape(1, s * d)

    blk = 1000
    assert n % blk == 0
    nb = n // blk
    const1 = lambda i: (0, 0)

    sw = pl.pallas_call(
        functools.partial(_selector_kernel, n_rows=float(n), n_blocks=nb),
        grid=(nb,),
        in_specs=[
            pl.BlockSpec((blk, d), lambda i: (i, 0)),
            pl.BlockSpec((d, dh), const1),
            pl.BlockSpec((1, dh), const1),
            pl.BlockSpec((dh, lanes), const1),
            pl.BlockSpec((1, lanes), const1),
        ],
        out_specs=pl.BlockSpec((1, lanes), const1),
        out_shape=jax.ShapeDtypeStruct((1, lanes), jnp.float32),
        compiler_params=pltpu.CompilerParams(
            dimension_semantics=("arbitrary",),
        ),
    )(x, W1, b1.reshape(1, dh), w2p, b2p)

    out = pl.pallas_call(
        functools.partial(_main_kernel, n_strategies=s, d=d),
        grid=(nb,),
        in_specs=[
            pl.BlockSpec((blk, d), lambda i: (i, 0)),
            pl.BlockSpec((1, lanes), const1),
            pl.BlockSpec((d, s * d), const1),
            pl.BlockSpec((1, s * d), const1),
            pl.BlockSpec((1, s * d), const1),
            pl.BlockSpec((1, s * d), const1),
            pl.BlockSpec((d, d), const1),
            pl.BlockSpec((1, d), const1),
            pl.BlockSpec((1, d), const1),
            pl.BlockSpec((1, d), const1),
        ],
        out_specs=pl.BlockSpec((blk, d), lambda i: (i, 0)),
        out_shape=jax.ShapeDtypeStruct((n, d), x.dtype),
        compiler_params=pltpu.CompilerParams(
            dimension_semantics=("arbitrary",),
        ),
    )(x, sw, wcat, bcat, gcat, betacat, wfc, bfc,
      gf.reshape(1, d), bf2.reshape(1, d))
    return out


# speedup vs baseline: 3.1672x; 1.0757x over previous
import functools

import jax
import jax.numpy as jnp
from jax.experimental import pallas as pl
from jax.experimental.pallas import tpu as pltpu

_EPS = 1e-5


def _selector_kernel(x_ref, w1_ref, b1_ref, w2p_ref, b2p_ref, sw_ref,
                     *, n_rows):
    gr = jnp.sum(x_ref[...], axis=0, keepdims=True) * (1.0 / n_rows)
    h = jnp.maximum(
        jnp.dot(gr, w1_ref[...], preferred_element_type=jnp.float32)
        + b1_ref[...], 0.0)
    logits = (jnp.dot(h, w2p_ref[...], preferred_element_type=jnp.float32)
              + b2p_ref[...])
    m = jnp.max(logits, axis=-1, keepdims=True)
    e = jnp.exp(logits - m)
    sw_ref[...] = e / jnp.sum(e, axis=-1, keepdims=True)


def _main_kernel(x_ref, sw_ref, wcat_ref, bcat_ref, gcat_ref, betacat_ref,
                 wf_ref, bf_ref, gf_ref, bf2_ref, out_ref,
                 *, n_strategies, d):
    xb = x_ref[...]
    h = (jnp.dot(xb, wcat_ref[...], preferred_element_type=jnp.float32)
         + bcat_ref[...])
    acc = jnp.zeros_like(xb)
    inv_d = 1.0 / d
    for s in range(n_strategies):
        sl = slice(s * d, (s + 1) * d)
        hs = h[:, sl]
        ss = jnp.sum(hs * hs, axis=-1, keepdims=True) * inv_d
        r = jax.lax.rsqrt(ss + _EPS)
        sw_s = sw_ref[0:1, s:s + 1]
        g = gcat_ref[:, sl] * sw_s
        beta = betacat_ref[:, sl] * sw_s
        acc = acc + jnp.maximum(hs * r * g + beta, 0.0)
    y = (jnp.dot(acc, wf_ref[...], preferred_element_type=jnp.float32)
         + bf_ref[...])
    ss = jnp.sum(y * y, axis=-1, keepdims=True) * inv_d
    r = jax.lax.rsqrt(ss + _EPS)
    out_ref[...] = jnp.maximum(y * r * gf_ref[...] + bf2_ref[...], 0.0)


def kernel(x, edge_index, W1, b1, W2, b2, Ws, bs, gs, betas, Wf, bf, gf, bf2):
    del edge_index
    n, d = x.shape
    s = Ws.shape[0]
    dh = W1.shape[1]
    lanes = 128

    w2p = jnp.zeros((dh, lanes), W2.dtype).at[:, :s].set(W2)
    b2p = jnp.full((1, lanes), -1e30, b2.dtype).at[0, :s].set(b2)
    wsc = Ws - jnp.mean(Ws, axis=2, keepdims=True)
    bsc = bs - jnp.mean(bs, axis=1, keepdims=True)
    wfc = Wf - jnp.mean(Wf, axis=1, keepdims=True)
    bfc = (bf - jnp.mean(bf)).reshape(1, d)
    wcat = jnp.transpose(wsc, (1, 0, 2)).reshape(d, s * d)
    bcat = bsc.reshape(1, s * d)
    gcat = gs.reshape(1, s * d)
    betacat = betas.reshape(1, s * d)

    blk = 1000
    assert n % blk == 0
    nb = n // blk
    const1 = lambda i: (0, 0)

    sw = pl.pallas_call(
        functools.partial(_selector_kernel, n_rows=float(n)),
        out_shape=jax.ShapeDtypeStruct((1, lanes), jnp.float32),
    )(x, W1, b1.reshape(1, dh), w2p, b2p)

    out = pl.pallas_call(
        functools.partial(_main_kernel, n_strategies=s, d=d),
        grid=(nb,),
        in_specs=[
            pl.BlockSpec((blk, d), lambda i: (i, 0)),
            pl.BlockSpec((1, lanes), const1),
            pl.BlockSpec((d, s * d), const1),
            pl.BlockSpec((1, s * d), const1),
            pl.BlockSpec((1, s * d), const1),
            pl.BlockSpec((1, s * d), const1),
            pl.BlockSpec((d, d), const1),
            pl.BlockSpec((1, d), const1),
            pl.BlockSpec((1, d), const1),
            pl.BlockSpec((1, d), const1),
        ],
        out_specs=pl.BlockSpec((blk, d), lambda i: (i, 0)),
        out_shape=jax.ShapeDtypeStruct((n, d), x.dtype),
        compiler_params=pltpu.CompilerParams(
            dimension_semantics=("arbitrary",),
        ),
    )(x, sw, wcat, bcat, gcat, betacat, wfc, bfc,
      gf.reshape(1, d), bf2.reshape(1, d))
    return out


# speedup vs baseline: 5.0985x; 1.6098x over previous
import functools

import jax
import jax.numpy as jnp
from jax.experimental import pallas as pl
from jax.experimental.pallas import tpu as pltpu

_EPS = 1e-5


def _prep_kernel(x_ref, w1_ref, b1_ref, w2_ref, b2_ref, ws_ref, bs_ref,
                 gs_ref, betas_ref, wf_ref, bf_ref,
                 sw_ref, wcat_ref, bcat_ref, gcat_ref, betacat_ref,
                 wfc_ref, bfc_ref, *, n_rows, n_strategies, d):
    gr = jnp.sum(x_ref[...], axis=0, keepdims=True) * (1.0 / n_rows)
    h = jnp.maximum(
        jnp.dot(gr, w1_ref[...], preferred_element_type=jnp.float32)
        + b1_ref[...], 0.0)
    logits = (jnp.dot(h, w2_ref[...], preferred_element_type=jnp.float32)
              + b2_ref[...])
    m = jnp.max(logits, axis=-1, keepdims=True)
    e = jnp.exp(logits - m)
    sm = e / jnp.sum(e, axis=-1, keepdims=True)
    sw_ref[...] = jnp.zeros_like(sw_ref)
    sw_ref[0:1, 0:n_strategies] = sm

    for s in range(n_strategies):
        sl = slice(s * d, (s + 1) * d)
        w = ws_ref[s]
        wcat_ref[:, sl] = w - jnp.mean(w, axis=1, keepdims=True)
        b = bs_ref[s:s + 1, :]
        bcat_ref[0:1, sl] = b - jnp.mean(b)
        sw_s = sm[0:1, s:s + 1]
        gcat_ref[0:1, sl] = gs_ref[s:s + 1, :] * sw_s
        betacat_ref[0:1, sl] = betas_ref[s:s + 1, :] * sw_s
    wf = wf_ref[...]
    wfc_ref[...] = wf - jnp.mean(wf, axis=1, keepdims=True)
    bfc_ref[...] = bf_ref[...] - jnp.mean(bf_ref[...])


def _main_kernel(x_ref, wcat_ref, bcat_ref, gcat_ref, betacat_ref,
                 wf_ref, bf_ref, gf_ref, bf2_ref, out_ref,
                 *, n_strategies, d):
    xb = x_ref[...]
    h = (jnp.dot(xb, wcat_ref[...], preferred_element_type=jnp.float32)
         + bcat_ref[...])
    acc = jnp.zeros_like(xb)
    inv_d = 1.0 / d
    for s in range(n_strategies):
        sl = slice(s * d, (s + 1) * d)
        hs = h[:, sl]
        ss = jnp.sum(hs * hs, axis=-1, keepdims=True) * inv_d
        r = jax.lax.rsqrt(ss + _EPS)
        acc = acc + jnp.maximum(hs * r * gcat_ref[:, sl] + betacat_ref[:, sl],
                                0.0)
    y = (jnp.dot(acc, wf_ref[...], preferred_element_type=jnp.float32)
         + bf_ref[...])
    ss = jnp.sum(y * y, axis=-1, keepdims=True) * inv_d
    r = jax.lax.rsqrt(ss + _EPS)
    out_ref[...] = jnp.maximum(y * r * gf_ref[...] + bf2_ref[...], 0.0)


def kernel(x, edge_index, W1, b1, W2, b2, Ws, bs, gs, betas, Wf, bf, gf, bf2):
    del edge_index
    n, d = x.shape
    s = Ws.shape[0]
    dh = W1.shape[1]
    lanes = 128

    f32 = jnp.float32
    sw, wcat, bcat, gcat, betacat, wfc, bfc = pl.pallas_call(
        functools.partial(_prep_kernel, n_rows=float(n), n_strategies=s, d=d),
        out_shape=(
            jax.ShapeDtypeStruct((1, lanes), f32),
            jax.ShapeDtypeStruct((d, s * d), f32),
            jax.ShapeDtypeStruct((1, s * d), f32),
            jax.ShapeDtypeStruct((1, s * d), f32),
            jax.ShapeDtypeStruct((1, s * d), f32),
            jax.ShapeDtypeStruct((d, d), f32),
            jax.ShapeDtypeStruct((1, d), f32),
        ),
    )(x, W1, b1.reshape(1, dh), W2, b2.reshape(1, s), Ws, bs, gs, betas,
      Wf, bf.reshape(1, d))
    del sw

    blk = 1000
    assert n % blk == 0
    nb = n // blk
    const1 = lambda i: (0, 0)

    out = pl.pallas_call(
        functools.partial(_main_kernel, n_strategies=s, d=d),
        grid=(nb,),
        in_specs=[
            pl.BlockSpec((blk, d), lambda i: (i, 0)),
            pl.BlockSpec((d, s * d), const1),
            pl.BlockSpec((1, s * d), const1),
            pl.BlockSpec((1, s * d), const1),
            pl.BlockSpec((1, s * d), const1),
            pl.BlockSpec((d, d), const1),
            pl.BlockSpec((1, d), const1),
            pl.BlockSpec((1, d), const1),
            pl.BlockSpec((1, d), const1),
        ],
        out_specs=pl.BlockSpec((blk, d), lambda i: (i, 0)),
        out_shape=jax.ShapeDtypeStruct((n, d), x.dtype),
        compiler_params=pltpu.CompilerParams(
            dimension_semantics=("arbitrary",),
        ),
    )(x, wcat, bcat, gcat, betacat, wfc, bfc,
      gf.reshape(1, d), bf2.reshape(1, d))
    return out


# speedup vs baseline: 5.5746x; 1.0934x over previous
import functools

import jax
import jax.numpy as jnp
from jax.experimental import pallas as pl
from jax.experimental.pallas import tpu as pltpu

_EPS = 1e-5


def _prep_kernel(x_ref, w1_ref, b1_ref, w2_ref, b2_ref, ws_ref, bs_ref,
                 gs_ref, betas_ref, wf_ref, bf_ref,
                 sw_ref, wcat_ref, bcat_ref, gcat_ref, betacat_ref,
                 wfc_ref, bfc_ref, *, n_rows, n_strategies, d):
    gr = jnp.sum(x_ref[...], axis=0, keepdims=True) * (1.0 / n_rows)
    h = jnp.maximum(
        jnp.dot(gr, w1_ref[...], preferred_element_type=jnp.float32)
        + b1_ref[...], 0.0)
    logits = (jnp.dot(h, w2_ref[...], preferred_element_type=jnp.float32)
              + b2_ref[...])
    m = jnp.max(logits, axis=-1, keepdims=True)
    e = jnp.exp(logits - m)
    sm = e / jnp.sum(e, axis=-1, keepdims=True)
    sw_ref[...] = jnp.zeros_like(sw_ref)
    sw_ref[0:1, 0:n_strategies] = sm

    for s in range(n_strategies):
        sl = slice(s * d, (s + 1) * d)
        w = ws_ref[s]
        wcat_ref[:, sl] = w - jnp.mean(w, axis=1, keepdims=True)
        b = bs_ref[s:s + 1, :]
        bcat_ref[0:1, sl] = b - jnp.mean(b)
        sw_s = sm[0:1, s:s + 1]
        gcat_ref[0:1, sl] = gs_ref[s:s + 1, :] * sw_s
        betacat_ref[0:1, sl] = betas_ref[s:s + 1, :] * sw_s
    wf = wf_ref[...]
    wfc_ref[...] = wf - jnp.mean(wf, axis=1, keepdims=True)
    bfc_ref[...] = bf_ref[...] - jnp.mean(bf_ref[...])


def _main_kernel(x_ref, wcat_ref, bcat_ref, gcat_ref, betacat_ref,
                 wf_ref, bf_ref, gf_ref, bf2_ref, out_ref,
                 *, n_strategies, d):
    xb = x_ref[...]
    h = (jnp.dot(xb, wcat_ref[...], preferred_element_type=jnp.float32)
         + bcat_ref[...])
    acc = jnp.zeros_like(xb)
    inv_d = 1.0 / d
    for s in range(n_strategies):
        sl = slice(s * d, (s + 1) * d)
        hs = h[:, sl]
        ss = jnp.sum(hs * hs, axis=-1, keepdims=True) * inv_d
        r = jax.lax.rsqrt(ss + _EPS)
        acc = acc + jnp.maximum(hs * r * gcat_ref[:, sl] + betacat_ref[:, sl],
                                0.0)
    y = (jnp.dot(acc, wf_ref[...], preferred_element_type=jnp.float32)
         + bf_ref[...])
    ss = jnp.sum(y * y, axis=-1, keepdims=True) * inv_d
    r = jax.lax.rsqrt(ss + _EPS)
    out_ref[...] = jnp.maximum(y * r * gf_ref[...] + bf2_ref[...], 0.0)


def kernel(x, edge_index, W1, b1, W2, b2, Ws, bs, gs, betas, Wf, bf, gf, bf2):
    del edge_index
    n, d = x.shape
    s = Ws.shape[0]
    dh = W1.shape[1]
    lanes = 128

    f32 = jnp.float32
    sw, wcat, bcat, gcat, betacat, wfc, bfc = pl.pallas_call(
        functools.partial(_prep_kernel, n_rows=float(n), n_strategies=s, d=d),
        out_shape=(
            jax.ShapeDtypeStruct((1, lanes), f32),
            jax.ShapeDtypeStruct((d, s * d), f32),
            jax.ShapeDtypeStruct((1, s * d), f32),
            jax.ShapeDtypeStruct((1, s * d), f32),
            jax.ShapeDtypeStruct((1, s * d), f32),
            jax.ShapeDtypeStruct((d, d), f32),
            jax.ShapeDtypeStruct((1, d), f32),
        ),
    )(x, W1, b1.reshape(1, dh), W2, b2.reshape(1, s), Ws, bs, gs, betas,
      Wf, bf.reshape(1, d))
    del sw

    blk = 2000
    assert n % blk == 0
    nb = n // blk
    const1 = lambda i: (0, 0)

    out = pl.pallas_call(
        functools.partial(_main_kernel, n_strategies=s, d=d),
        grid=(nb,),
        in_specs=[
            pl.BlockSpec((blk, d), lambda i: (i, 0)),
            pl.BlockSpec((d, s * d), const1),
            pl.BlockSpec((1, s * d), const1),
            pl.BlockSpec((1, s * d), const1),
            pl.BlockSpec((1, s * d), const1),
            pl.BlockSpec((d, d), const1),
            pl.BlockSpec((1, d), const1),
            pl.BlockSpec((1, d), const1),
            pl.BlockSpec((1, d), const1),
        ],
        out_specs=pl.BlockSpec((blk, d), lambda i: (i, 0)),
        out_shape=jax.ShapeDtypeStruct((n, d), x.dtype),
        compiler_params=pltpu.CompilerParams(
            dimension_semantics=("arbitrary",),
        ),
    )(x, wcat, bcat, gcat, betacat, wfc, bfc,
      gf.reshape(1, d), bf2.reshape(1, d))
    return out
